# baseline (speedup 1.0000x reference)
import functools

import jax
import jax.numpy as jnp
import numpy as np
from jax import lax
from jax.experimental import pallas as pl
from jax.experimental.pallas import tpu as pltpu
from jax.experimental.pallas import tpu_sc as plsc

MAX_LEN = 200
EMB = 64
NC = 2
NS = 16
NW = NC * NS
CHUNK = 128


def _positional_signal_np(hidden_size: int, length: int) -> np.ndarray:
    position = np.arange(0, length, dtype=np.float32)
    num_timescales = hidden_size // 2
    log_inc = np.float32(np.log(10000.0) / (num_timescales - 1))
    inv_timescales = np.exp(np.arange(num_timescales, dtype=np.float32) * -log_inc)
    scaled = position[:, None] * inv_timescales[None, :]
    return np.concatenate([np.sin(scaled), np.cos(scaled)], axis=1).astype(np.float32)


def _make_kernel(total_rows: int):
    rows_per_w = total_rows // NW
    n_chunks = rows_per_w // CHUNK
    mesh = plsc.VectorSubcoreMesh(core_axis_name="c", subcore_axis_name="s")

    @functools.partial(
        pl.kernel,
        mesh=mesh,
        compiler_params=pltpu.CompilerParams(use_tc_tiling_on_sc=False),
        out_type=jax.ShapeDtypeStruct((total_rows, EMB), jnp.float32),
        scratch_types=[
            pltpu.VMEM((CHUNK,), jnp.int32),
            pltpu.VMEM((CHUNK, EMB), jnp.float32),
            pltpu.VMEM((2 * MAX_LEN * EMB,), jnp.float32),
            pltpu.SemaphoreType.DMA,
        ],
    )
    def k(ids_hbm, sig_hbm, table_hbm, out_hbm, idx_v, rows_v, sig_v, sem):
        wid = lax.axis_index("s") * NC + lax.axis_index("c")
        base = wid * rows_per_w
        pltpu.sync_copy(sig_hbm, sig_v)

        def chunk_body(c, carry):
            row0 = base + c * CHUNK
            pltpu.sync_copy(ids_hbm.at[pl.ds(row0, CHUNK)], idx_v)
            pltpu.async_copy(table_hbm.at[idx_v], rows_v, sem).wait()
            soff = (c * CHUNK) % MAX_LEN * EMB

            def row_body(i, _):
                o = soff + i * EMB
                for j in range(EMB // 16):
                    rows_v[i, pl.ds(16 * j, 16)] = (
                        rows_v[i, pl.ds(16 * j, 16)] + sig_v[pl.ds(o + 16 * j, 16)]
                    )
                return 0

            lax.fori_loop(0, CHUNK, row_body, 0)
            pltpu.sync_copy(rows_v, out_hbm.at[pl.ds(row0, CHUNK)])
            return carry

        lax.fori_loop(0, n_chunks, chunk_body, 0)

    return k


def kernel(x, table):
    b, l = x.shape
    ids = x.reshape(-1).astype(jnp.int32)
    sig = _positional_signal_np(EMB, MAX_LEN)
    sig2 = jnp.asarray(np.concatenate([sig, sig], axis=0).reshape(-1))
    out = _make_kernel(b * l)(ids, sig2, table)
    return out.reshape(b, l, EMB)


# speedup vs baseline: 1.6014x; 1.6014x over previous
import functools

import jax
import jax.numpy as jnp
import numpy as np
from jax import lax
from jax.experimental import pallas as pl
from jax.experimental.pallas import tpu as pltpu
from jax.experimental.pallas import tpu_sc as plsc

MAX_LEN = 200
EMB = 64
NC = 2
NS = 16
NW = NC * NS
CHUNK = 128
NBUF = 4


def _positional_signal_np(hidden_size: int, length: int) -> np.ndarray:
    position = np.arange(0, length, dtype=np.float32)
    num_timescales = hidden_size // 2
    log_inc = np.float32(np.log(10000.0) / (num_timescales - 1))
    inv_timescales = np.exp(np.arange(num_timescales, dtype=np.float32) * -log_inc)
    scaled = position[:, None] * inv_timescales[None, :]
    return np.concatenate([np.sin(scaled), np.cos(scaled)], axis=1).astype(np.float32)


def _make_kernel(total_rows: int):
    rows_per_w = total_rows // NW
    n_chunks = rows_per_w // CHUNK
    n_macro = n_chunks // NBUF
    mesh = plsc.VectorSubcoreMesh(core_axis_name="c", subcore_axis_name="s")

    @functools.partial(
        pl.kernel,
        mesh=mesh,
        compiler_params=pltpu.CompilerParams(use_tc_tiling_on_sc=False),
        out_type=jax.ShapeDtypeStruct((total_rows, EMB), jnp.float32),
        scratch_types=[
            pltpu.VMEM((n_chunks, CHUNK), jnp.int32),
            pltpu.VMEM((2 * MAX_LEN * EMB,), jnp.float32),
        ]
        + [pltpu.VMEM((CHUNK, EMB), jnp.float32)] * NBUF
        + [pltpu.SemaphoreType.DMA] * (2 * NBUF),
    )
    def k(ids_hbm, sig_hbm, table_hbm, out_hbm, idx_v, sig_v, *bufs):
        rows = bufs[:NBUF]
        gsem = bufs[NBUF : 2 * NBUF]
        ssem = bufs[2 * NBUF :]
        wid = lax.axis_index("s") * NC + lax.axis_index("c")
        base = wid * rows_per_w
        pltpu.sync_copy(sig_hbm, sig_v)
        pltpu.sync_copy(ids_hbm.at[wid], idx_v)

        def start_gather(c, b):
            pltpu.async_copy(table_hbm.at[idx_v.at[c]], rows[b], gsem[b])

        def wait_gather(c, b):
            pltpu.make_async_copy(table_hbm.at[idx_v.at[c]], rows[b], gsem[b]).wait()

        def start_store(c, b):
            pltpu.async_copy(rows[b], out_hbm.at[pl.ds(base + c * CHUNK, CHUNK)], ssem[b])

        def wait_store(c, b):
            pltpu.make_async_copy(
                rows[b], out_hbm.at[pl.ds(base + c * CHUNK, CHUNK)], ssem[b]
            ).wait()

        def process(m, b, wait_prev_store, prefetch):
            c = m * NBUF + b
            wait_gather(c, b)
            soff = lax.rem(c * CHUNK, MAX_LEN) * EMB

            @plsc.parallel_loop(0, CHUNK, 1, unroll=4)
            def _row(i):
                o = soff + i * EMB
                for j in range(EMB // 16):
                    rows[b][i, pl.ds(16 * j, 16)] = (
                        rows[b][i, pl.ds(16 * j, 16)] + sig_v[pl.ds(o + 16 * j, 16)]
                    )

            start_store(c, b)
            if prefetch:
                bg = (b - 1) % NBUF
                if wait_prev_store:
                    wait_store(c - 1, bg)
                start_gather(c + NBUF - 1, bg)

        for b in range(NBUF - 1):
            start_gather(b, b)
        for b in range(NBUF):
            process(0, b, wait_prev_store=(b > 0), prefetch=True)

        def macro(m, carry):
            for b in range(NBUF):
                process(m, b, wait_prev_store=True, prefetch=True)
            return carry

        lax.fori_loop(1, n_macro - 1, macro, 0)
        for b in range(NBUF):
            process(n_macro - 1, b, wait_prev_store=(b == 0), prefetch=(b == 0))
        for b in range(NBUF):
            wait_store(n_chunks - NBUF + b, b)

    return k


def kernel(x, table):
    b, l = x.shape
    total = b * l
    ids = x.reshape(NW, (total // NW) // CHUNK, CHUNK).astype(jnp.int32)
    sig = _positional_signal_np(EMB, MAX_LEN)
    sig2 = jnp.asarray(np.concatenate([sig, sig], axis=0).reshape(-1))
    out = _make_kernel(total)(ids, sig2, table)
    return out.reshape(b, l, EMB)
